# baseline (speedup 1.0000x reference)
import functools

import jax
import jax.numpy as jnp
from jax import lax
from jax.experimental import pallas as pl
from jax.experimental.pallas import tpu as pltpu
from jax.experimental.pallas import tpu_sc as plsc

N_ROWS = 50000
D = 256
NSEG = 128
CHUNK = 80
NBLK = N_ROWS // CHUNK
NC = 2
NS = 16
NW = NC * NS
SUMW = NSEG * D
CNTW = NSEG * 16
MAXB = NBLK // NW + 1
NBUF = 4


def _sc_segment_sums(x, batch):
  mesh = plsc.VectorSubcoreMesh(core_axis_name="c", subcore_axis_name="s")

  @functools.partial(
      pl.kernel,
      mesh=mesh,
      out_type=(
          jax.ShapeDtypeStruct((NW, SUMW), jnp.float32),
          jax.ShapeDtypeStruct((NW, CNTW), jnp.float32),
      ),
      scratch_types=[
          pltpu.VMEM((MAXB * CHUNK,), jnp.int32),
          *[pltpu.VMEM((CHUNK, D), jnp.float32) for _ in range(NBUF)],
          pltpu.VMEM((SUMW,), jnp.float32),
          pltpu.VMEM((CNTW,), jnp.float32),
          *[pltpu.SemaphoreType.DMA for _ in range(NBUF)],
          pltpu.SemaphoreType.DMA,
      ],
  )
  def k(x_hbm, b_hbm, sums_hbm, cnts_hbm,
        idx_all, xb0, xb1, xb2, xb3, acc_v, cnt_v,
        sem0, sem1, sem2, sem3, semi):
    bufs = (xb0, xb1, xb2, xb3)
    sems = (sem0, sem1, sem2, sem3)
    c = lax.axis_index("c")
    s = lax.axis_index("s")
    w = c * NS + s

    one16 = jnp.ones((16,), jnp.float32)
    chunk16 = jnp.full((16,), float(CHUNK), jnp.float32)
    zero16 = jnp.zeros((16,), jnp.float32)
    lo = w * NBLK // NW
    hi = (w + 1) * NBLK // NW

    pltpu.async_copy(b_hbm.at[pl.ds(lo * CHUNK, MAXB * CHUNK)], idx_all, semi)

    def issue(b, xbuf, sem):
      pltpu.async_copy(x_hbm.at[pl.ds(b * CHUNK, CHUNK)], xbuf, sem)

    def drain(xbuf, sem):
      pltpu.make_async_copy(x_hbm.at[pl.ds(0, CHUNK)], xbuf, sem).wait()

    for kk in range(NBUF):
      issue(lo + kk, bufs[kk], sems[kk])

    def zsum_body(i, carry):
      for u in range(8):
        acc_v[pl.ds(i * 128 + u * 16, 16)] = zero16
      return carry

    lax.fori_loop(0, SUMW // 128, zsum_body, 0)

    def zcnt_body(i, carry):
      for u in range(8):
        cnt_v[pl.ds(i * 128 + u * 16, 16)] = zero16
      return carry

    lax.fori_loop(0, CNTW // 128, zcnt_body, 0)

    pltpu.make_async_copy(b_hbm.at[pl.ds(0, MAXB * CHUNK)], idx_all, semi
                          ).wait()

    def compute(b, xbuf):
      ioff = (b - lo) * CHUNK
      segs_first = idx_all[pl.ds(ioff, 16)]
      segs_last = idx_all[pl.ds(ioff + CHUNK - 16, 16)]
      seg_a = segs_first[0]
      seg_b = segs_last[15]

      @pl.when(seg_a == seg_b)
      def _():
        offa = seg_a * D

        def gsum(g, carry2):
          for j in range(D // 16):
            s0 = xbuf[g * 16, pl.ds(j * 16, 16)]
            for r in range(1, 16):
              s0 = s0 + xbuf[g * 16 + r, pl.ds(j * 16, 16)]
            plsc.addupdate(acc_v.at[pl.ds(offa + j * 16, 16)], s0)
          return carry2

        lax.fori_loop(0, CHUNK // 16, gsum, 0)
        plsc.addupdate(cnt_v.at[pl.ds(seg_a * 16, 16)], chunk16)

      @pl.when(seg_a != seg_b)
      def _():
        compute_rows(ioff, xbuf)

    def compute_rows(ioff, xbuf):
      def group_body(g, carry2):
        segs = idx_all[pl.ds(ioff + g * 16, 16)]
        for jj in range(0, 16, 2):
          sa = segs[jj]
          sb = segs[jj + 1]
          offa = sa * D
          offb = sb * D
          ra = g * 16 + jj
          rb = ra + 1
          va = [xbuf[ra, pl.ds(j * 16, 16)] for j in range(D // 16)]
          vb = [xbuf[rb, pl.ds(j * 16, 16)] for j in range(D // 16)]
          for j in range(D // 16):
            plsc.addupdate(acc_v.at[pl.ds(offa + j * 16, 16)], va[j])
          plsc.addupdate(cnt_v.at[pl.ds(sa * 16, 16)], one16)
          for j in range(D // 16):
            plsc.addupdate(acc_v.at[pl.ds(offb + j * 16, 16)], vb[j])
          plsc.addupdate(cnt_v.at[pl.ds(sb * 16, 16)], one16)
        return carry2

      lax.fori_loop(0, CHUNK // 16, group_body, 0)

    def ring_body(q, carry):
      for kk in range(NBUF):
        b = lo + q * NBUF + kk

        @pl.when(b < hi)
        def _():
          drain(bufs[kk], sems[kk])
          compute(b, bufs[kk])

          @pl.when(b + NBUF < hi)
          def _():
            issue(b + NBUF, bufs[kk], sems[kk])

      return carry

    lax.fori_loop(0, (MAXB + NBUF - 1) // NBUF, ring_body, 0)

    pltpu.sync_copy(acc_v, sums_hbm.at[w])
    pltpu.sync_copy(cnt_v, cnts_hbm.at[w])

  return k(x, batch)


def _combine(sums, cnts):
  def body(s_ref, c_ref, o_ref):
    ssum = jnp.sum(s_ref[...], axis=0)
    cc = jnp.sum(c_ref[...][:, :, 0:1], axis=0)
    o_ref[...] = ssum / jnp.maximum(cc, 1.0)

  return pl.pallas_call(
      body,
      out_shape=jax.ShapeDtypeStruct((NSEG, D), jnp.float32),
  )(sums, cnts)


@jax.jit
def kernel(x, batch):
  batch = batch.astype(jnp.int32)
  sums, cnts = _sc_segment_sums(x, batch)
  return _combine(sums.reshape(NW, NSEG, D), cnts.reshape(NW, NSEG, 16))


# speedup vs baseline: 1.3783x; 1.3783x over previous
import functools

import jax
import jax.numpy as jnp
from jax import lax
from jax.experimental import pallas as pl
from jax.experimental.pallas import tpu as pltpu
from jax.experimental.pallas import tpu_sc as plsc

N_ROWS = 50000
D = 256
NSEG = 128

TBLK = 512
NTC = 23040
G = NTC // TBLK

BASE = NTC
NSC = N_ROWS - NTC
CHUNK = 80
NBLK = NSC // CHUNK
NC = 2
NS = 16
NW = NC * NS
SUMW = NSEG * D
CNTW = NSEG * 16
MAXB = NBLK // NW + 1
NBUF = 4


def _sc_segment_sums(x, batch):
  mesh = plsc.VectorSubcoreMesh(core_axis_name="c", subcore_axis_name="s")

  @functools.partial(
      pl.kernel,
      mesh=mesh,
      out_type=(
          jax.ShapeDtypeStruct((NW, SUMW), jnp.float32),
          jax.ShapeDtypeStruct((NW, CNTW), jnp.float32),
      ),
      scratch_types=[
          pltpu.VMEM((MAXB * CHUNK,), jnp.int32),
          *[pltpu.VMEM((CHUNK, D), jnp.float32) for _ in range(NBUF)],
          pltpu.VMEM((SUMW,), jnp.float32),
          pltpu.VMEM((CNTW,), jnp.float32),
          *[pltpu.SemaphoreType.DMA for _ in range(NBUF)],
          pltpu.SemaphoreType.DMA,
      ],
  )
  def k(x_hbm, b_hbm, sums_hbm, cnts_hbm,
        idx_all, xb0, xb1, xb2, xb3, acc_v, cnt_v,
        sem0, sem1, sem2, sem3, semi):
    bufs = (xb0, xb1, xb2, xb3)
    sems = (sem0, sem1, sem2, sem3)
    c = lax.axis_index("c")
    s = lax.axis_index("s")
    w = c * NS + s

    one16 = jnp.ones((16,), jnp.float32)
    zero16 = jnp.zeros((16,), jnp.float32)
    lo = w * NBLK // NW
    hi = (w + 1) * NBLK // NW

    pltpu.async_copy(
        b_hbm.at[pl.ds(BASE + lo * CHUNK, MAXB * CHUNK)], idx_all, semi)

    def issue(b, xbuf, sem):
      pltpu.async_copy(x_hbm.at[pl.ds(BASE + b * CHUNK, CHUNK)], xbuf, sem)

    def drain(xbuf, sem):
      pltpu.make_async_copy(x_hbm.at[pl.ds(0, CHUNK)], xbuf, sem).wait()

    for kk in range(NBUF):
      issue(lo + kk, bufs[kk], sems[kk])

    def zsum_body(i, carry):
      for u in range(8):
        acc_v[pl.ds(i * 128 + u * 16, 16)] = zero16
      return carry

    lax.fori_loop(0, SUMW // 128, zsum_body, 0)

    def zcnt_body(i, carry):
      for u in range(8):
        cnt_v[pl.ds(i * 128 + u * 16, 16)] = zero16
      return carry

    lax.fori_loop(0, CNTW // 128, zcnt_body, 0)

    pltpu.make_async_copy(b_hbm.at[pl.ds(0, MAXB * CHUNK)], idx_all, semi
                          ).wait()

    def compute(b, xbuf):
      ioff = (b - lo) * CHUNK

      def group_body(g, carry2):
        segs = idx_all[pl.ds(ioff + g * 16, 16)]
        for jj in range(0, 16, 2):
          sa = segs[jj]
          sb = segs[jj + 1]
          offa = sa * D
          offb = sb * D
          ra = g * 16 + jj
          rb = ra + 1
          va = [xbuf[ra, pl.ds(j * 16, 16)] for j in range(D // 16)]
          vb = [xbuf[rb, pl.ds(j * 16, 16)] for j in range(D // 16)]
          for j in range(D // 16):
            plsc.addupdate(acc_v.at[pl.ds(offa + j * 16, 16)], va[j])
          plsc.addupdate(cnt_v.at[pl.ds(sa * 16, 16)], one16)
          for j in range(D // 16):
            plsc.addupdate(acc_v.at[pl.ds(offb + j * 16, 16)], vb[j])
          plsc.addupdate(cnt_v.at[pl.ds(sb * 16, 16)], one16)
        return carry2

      lax.fori_loop(0, CHUNK // 16, group_body, 0)

    def ring_body(q, carry):
      for kk in range(NBUF):
        b = lo + q * NBUF + kk

        @pl.when(b < hi)
        def _():
          drain(bufs[kk], sems[kk])
          compute(b, bufs[kk])

          @pl.when(b + NBUF < hi)
          def _():
            issue(b + NBUF, bufs[kk], sems[kk])

      return carry

    lax.fori_loop(0, (MAXB + NBUF - 1) // NBUF, ring_body, 0)

    pltpu.sync_copy(acc_v, sums_hbm.at[w])
    pltpu.sync_copy(cnt_v, cnts_hbm.at[w])

  return k(x, batch)


def _tc_segment_sums(x, batch_tc):
  def body(b_ref, x_ref, o_ref, c_ref):
    i = pl.program_id(0)
    seg = b_ref[0, 0, :]
    oh = (seg[:, None]
          == lax.broadcasted_iota(jnp.int32, (TBLK, NSEG), 1)
          ).astype(jnp.float32)
    ps = lax.dot_general(oh, x_ref[...], (((0,), (0,)), ((), ())),
                         preferred_element_type=jnp.float32)
    pc = jnp.sum(oh, axis=0)[None, :]

    @pl.when(i == 0)
    def _():
      o_ref[...] = jnp.zeros_like(o_ref)
      c_ref[...] = jnp.zeros_like(c_ref)

    o_ref[...] += ps
    c_ref[...] += pc

  return pl.pallas_call(
      body,
      grid=(G,),
      in_specs=[
          pl.BlockSpec((1, 1, TBLK), lambda i: (i, 0, 0)),
          pl.BlockSpec((TBLK, D), lambda i: (i, 0)),
      ],
      out_specs=[
          pl.BlockSpec((NSEG, D), lambda i: (0, 0)),
          pl.BlockSpec((1, NSEG), lambda i: (0, 0)),
      ],
      out_shape=(
          jax.ShapeDtypeStruct((NSEG, D), jnp.float32),
          jax.ShapeDtypeStruct((1, NSEG), jnp.float32),
      ),
  )(batch_tc, x)


def _combine(sums, cnts, tsum, tcnt):
  def body(s_ref, c_ref, ts_ref, tc_ref, o_ref):
    ssum = jnp.sum(s_ref[...], axis=0) + ts_ref[...]
    cc = jnp.sum(c_ref[...][:, :, 0], axis=0) + tc_ref[0]
    o_ref[...] = ssum / jnp.maximum(cc, 1.0)[:, None]

  return pl.pallas_call(
      body,
      out_shape=jax.ShapeDtypeStruct((NSEG, D), jnp.float32),
  )(sums, cnts, tsum, tcnt)


@jax.jit
def kernel(x, batch):
  batch = batch.astype(jnp.int32)
  batch_tc = batch[:NTC].reshape(G, 1, TBLK)
  sums, cnts = _sc_segment_sums(x, batch)
  tsum, tcnt = _tc_segment_sums(x, batch_tc)
  return _combine(sums.reshape(NW, NSEG, D), cnts.reshape(NW, NSEG, 16),
                  tsum, tcnt)


# speedup vs baseline: 1.4570x; 1.0571x over previous
import functools

import jax
import jax.numpy as jnp
from jax import lax
from jax.experimental import pallas as pl
from jax.experimental.pallas import tpu as pltpu
from jax.experimental.pallas import tpu_sc as plsc

N_ROWS = 50000
D = 256
NSEG = 128

TBLK = 2304
NTC = 23040
G = NTC // TBLK

BASE = NTC
NSC = N_ROWS - NTC
CHUNK = 80
NBLK = NSC // CHUNK
NC = 2
NS = 16
NW = NC * NS
SUMW = NSEG * D
CNTW = NSEG * 16
MAXB = NBLK // NW + 1
NBUF = 4


def _sc_segment_sums(x, batch):
  mesh = plsc.VectorSubcoreMesh(core_axis_name="c", subcore_axis_name="s")

  @functools.partial(
      pl.kernel,
      mesh=mesh,
      out_type=(
          jax.ShapeDtypeStruct((NW, SUMW), jnp.float32),
          jax.ShapeDtypeStruct((NW, CNTW), jnp.float32),
      ),
      scratch_types=[
          pltpu.VMEM((MAXB * CHUNK,), jnp.int32),
          *[pltpu.VMEM((CHUNK, D), jnp.float32) for _ in range(NBUF)],
          pltpu.VMEM((SUMW,), jnp.float32),
          pltpu.VMEM((CNTW,), jnp.float32),
          *[pltpu.SemaphoreType.DMA for _ in range(NBUF)],
          pltpu.SemaphoreType.DMA,
      ],
  )
  def k(x_hbm, b_hbm, sums_hbm, cnts_hbm,
        idx_all, xb0, xb1, xb2, xb3, acc_v, cnt_v,
        sem0, sem1, sem2, sem3, semi):
    bufs = (xb0, xb1, xb2, xb3)
    sems = (sem0, sem1, sem2, sem3)
    c = lax.axis_index("c")
    s = lax.axis_index("s")
    w = c * NS + s

    one16 = jnp.ones((16,), jnp.float32)
    zero16 = jnp.zeros((16,), jnp.float32)
    lo = w * NBLK // NW
    hi = (w + 1) * NBLK // NW

    pltpu.async_copy(
        b_hbm.at[pl.ds(BASE + lo * CHUNK, MAXB * CHUNK)], idx_all, semi)

    def issue(b, xbuf, sem):
      pltpu.async_copy(x_hbm.at[pl.ds(BASE + b * CHUNK, CHUNK)], xbuf, sem)

    def drain(xbuf, sem):
      pltpu.make_async_copy(x_hbm.at[pl.ds(0, CHUNK)], xbuf, sem).wait()

    for kk in range(NBUF):
      issue(lo + kk, bufs[kk], sems[kk])

    def zsum_body(i, carry):
      for u in range(8):
        acc_v[pl.ds(i * 128 + u * 16, 16)] = zero16
      return carry

    lax.fori_loop(0, SUMW // 128, zsum_body, 0)

    def zcnt_body(i, carry):
      for u in range(8):
        cnt_v[pl.ds(i * 128 + u * 16, 16)] = zero16
      return carry

    lax.fori_loop(0, CNTW // 128, zcnt_body, 0)

    pltpu.make_async_copy(b_hbm.at[pl.ds(0, MAXB * CHUNK)], idx_all, semi
                          ).wait()

    def compute(b, xbuf):
      ioff = (b - lo) * CHUNK

      def group_body(g, carry2):
        segs = idx_all[pl.ds(ioff + g * 16, 16)]
        for jj in range(0, 16, 2):
          sa = segs[jj]
          sb = segs[jj + 1]
          offa = sa * D
          offb = sb * D
          ra = g * 16 + jj
          rb = ra + 1
          va = [xbuf[ra, pl.ds(j * 16, 16)] for j in range(D // 16)]
          vb = [xbuf[rb, pl.ds(j * 16, 16)] for j in range(D // 16)]
          for j in range(D // 16):
            plsc.addupdate(acc_v.at[pl.ds(offa + j * 16, 16)], va[j])
          plsc.addupdate(cnt_v.at[pl.ds(sa * 16, 16)], one16)
          for j in range(D // 16):
            plsc.addupdate(acc_v.at[pl.ds(offb + j * 16, 16)], vb[j])
          plsc.addupdate(cnt_v.at[pl.ds(sb * 16, 16)], one16)
        return carry2

      lax.fori_loop(0, CHUNK // 16, group_body, 0)

    def ring_body(q, carry):
      for kk in range(NBUF):
        b = lo + q * NBUF + kk

        @pl.when(b < hi)
        def _():
          drain(bufs[kk], sems[kk])
          compute(b, bufs[kk])

          @pl.when(b + NBUF < hi)
          def _():
            issue(b + NBUF, bufs[kk], sems[kk])

      return carry

    lax.fori_loop(0, (MAXB + NBUF - 1) // NBUF, ring_body, 0)

    pltpu.sync_copy(acc_v, sums_hbm.at[w])
    pltpu.sync_copy(cnt_v, cnts_hbm.at[w])

  return k(x, batch)


def _tc_segment_sums(x, batch_tc):
  def body(b_ref, x_ref, o_ref, c_ref):
    i = pl.program_id(0)
    seg = b_ref[0, 0, :]
    oh = (seg[:, None]
          == lax.broadcasted_iota(jnp.int32, (TBLK, NSEG), 1)
          ).astype(jnp.float32)
    ps = lax.dot_general(oh, x_ref[...], (((0,), (0,)), ((), ())),
                         preferred_element_type=jnp.float32)
    pc = jnp.sum(oh, axis=0)[None, :]

    @pl.when(i == 0)
    def _():
      o_ref[...] = jnp.zeros_like(o_ref)
      c_ref[...] = jnp.zeros_like(c_ref)

    o_ref[...] += ps
    c_ref[...] += pc

  return pl.pallas_call(
      body,
      grid=(G,),
      in_specs=[
          pl.BlockSpec((1, 1, TBLK), lambda i: (i, 0, 0)),
          pl.BlockSpec((TBLK, D), lambda i: (i, 0)),
      ],
      out_specs=[
          pl.BlockSpec((NSEG, D), lambda i: (0, 0)),
          pl.BlockSpec((1, NSEG), lambda i: (0, 0)),
      ],
      out_shape=(
          jax.ShapeDtypeStruct((NSEG, D), jnp.float32),
          jax.ShapeDtypeStruct((1, NSEG), jnp.float32),
      ),
  )(batch_tc, x)


def _combine(sums, cnts, tsum, tcnt):
  def body(s_ref, c_ref, ts_ref, tc_ref, o_ref):
    ssum = jnp.sum(s_ref[...], axis=0) + ts_ref[...]
    cc = jnp.sum(c_ref[...][:, :, 0], axis=0) + tc_ref[0]
    o_ref[...] = ssum / jnp.maximum(cc, 1.0)[:, None]

  return pl.pallas_call(
      body,
      out_shape=jax.ShapeDtypeStruct((NSEG, D), jnp.float32),
  )(sums, cnts, tsum, tcnt)


@jax.jit
def kernel(x, batch):
  batch = batch.astype(jnp.int32)
  batch_tc = batch[:NTC].reshape(G, 1, TBLK)
  tsum, tcnt = _tc_segment_sums(x, batch_tc)
  sums, cnts = _sc_segment_sums(x, batch)
  return _combine(sums.reshape(NW, NSEG, D), cnts.reshape(NW, NSEG, 16),
                  tsum, tcnt)


# speedup vs baseline: 1.5339x; 1.0527x over previous
import functools

import jax
import jax.numpy as jnp
from jax import lax
from jax.experimental import pallas as pl
from jax.experimental.pallas import tpu as pltpu
from jax.experimental.pallas import tpu_sc as plsc

N_ROWS = 50000
D = 256
NSEG = 128

TBLK = 2000
NTC = 28000
G = NTC // TBLK

BASE = NTC
NSC = N_ROWS - NTC
CHUNK = 80
NBLK = NSC // CHUNK
NC = 2
NS = 16
NW = NC * NS
SUMW = NSEG * D
CNTW = NSEG * 16
MAXB = NBLK // NW + 1
NBUF = 4


def _sc_segment_sums(x, batch):
  mesh = plsc.VectorSubcoreMesh(core_axis_name="c", subcore_axis_name="s")

  @functools.partial(
      pl.kernel,
      mesh=mesh,
      out_type=(
          jax.ShapeDtypeStruct((NW, SUMW), jnp.float32),
          jax.ShapeDtypeStruct((NW, CNTW), jnp.float32),
      ),
      scratch_types=[
          pltpu.VMEM((MAXB * CHUNK,), jnp.int32),
          *[pltpu.VMEM((CHUNK, D), jnp.float32) for _ in range(NBUF)],
          pltpu.VMEM((SUMW,), jnp.float32),
          pltpu.VMEM((CNTW,), jnp.float32),
          *[pltpu.SemaphoreType.DMA for _ in range(NBUF)],
          pltpu.SemaphoreType.DMA,
      ],
  )
  def k(x_hbm, b_hbm, sums_hbm, cnts_hbm,
        idx_all, xb0, xb1, xb2, xb3, acc_v, cnt_v,
        sem0, sem1, sem2, sem3, semi):
    bufs = (xb0, xb1, xb2, xb3)
    sems = (sem0, sem1, sem2, sem3)
    c = lax.axis_index("c")
    s = lax.axis_index("s")
    w = c * NS + s

    one16 = jnp.ones((16,), jnp.float32)
    zero16 = jnp.zeros((16,), jnp.float32)
    lo = w * NBLK // NW
    hi = (w + 1) * NBLK // NW

    pltpu.async_copy(
        b_hbm.at[pl.ds(BASE + lo * CHUNK, MAXB * CHUNK)], idx_all, semi)

    def issue(b, xbuf, sem):
      pltpu.async_copy(x_hbm.at[pl.ds(BASE + b * CHUNK, CHUNK)], xbuf, sem)

    def drain(xbuf, sem):
      pltpu.make_async_copy(x_hbm.at[pl.ds(0, CHUNK)], xbuf, sem).wait()

    for kk in range(NBUF):
      issue(lo + kk, bufs[kk], sems[kk])

    def zsum_body(i, carry):
      for u in range(8):
        acc_v[pl.ds(i * 128 + u * 16, 16)] = zero16
      return carry

    lax.fori_loop(0, SUMW // 128, zsum_body, 0)

    def zcnt_body(i, carry):
      for u in range(8):
        cnt_v[pl.ds(i * 128 + u * 16, 16)] = zero16
      return carry

    lax.fori_loop(0, CNTW // 128, zcnt_body, 0)

    pltpu.make_async_copy(b_hbm.at[pl.ds(0, MAXB * CHUNK)], idx_all, semi
                          ).wait()

    def compute(b, xbuf):
      ioff = (b - lo) * CHUNK

      def group_body(g, carry2):
        segs = idx_all[pl.ds(ioff + g * 16, 16)]
        for jj in range(0, 16, 2):
          sa = segs[jj]
          sb = segs[jj + 1]
          offa = sa * D
          offb = sb * D
          ra = g * 16 + jj
          rb = ra + 1
          va = [xbuf[ra, pl.ds(j * 16, 16)] for j in range(D // 16)]
          vb = [xbuf[rb, pl.ds(j * 16, 16)] for j in range(D // 16)]
          for j in range(D // 16):
            plsc.addupdate(acc_v.at[pl.ds(offa + j * 16, 16)], va[j])
          plsc.addupdate(cnt_v.at[pl.ds(sa * 16, 16)], one16)
          for j in range(D // 16):
            plsc.addupdate(acc_v.at[pl.ds(offb + j * 16, 16)], vb[j])
          plsc.addupdate(cnt_v.at[pl.ds(sb * 16, 16)], one16)
        return carry2

      lax.fori_loop(0, CHUNK // 16, group_body, 0)

    def ring_body(q, carry):
      for kk in range(NBUF):
        b = lo + q * NBUF + kk

        @pl.when(b < hi)
        def _():
          drain(bufs[kk], sems[kk])
          compute(b, bufs[kk])

          @pl.when(b + NBUF < hi)
          def _():
            issue(b + NBUF, bufs[kk], sems[kk])

      return carry

    lax.fori_loop(0, (MAXB + NBUF - 1) // NBUF, ring_body, 0)

    pltpu.sync_copy(acc_v, sums_hbm.at[w])
    pltpu.sync_copy(cnt_v, cnts_hbm.at[w])

  return k(x, batch)


def _tc_segment_sums(x, batch_tc):
  def body(b_ref, x_ref, o_ref, c_ref):
    i = pl.program_id(0)
    seg = b_ref[0, 0, :]
    oh = (seg[:, None]
          == lax.broadcasted_iota(jnp.int32, (TBLK, NSEG), 1)
          ).astype(jnp.float32)
    ps = lax.dot_general(oh, x_ref[...], (((0,), (0,)), ((), ())),
                         preferred_element_type=jnp.float32)
    pc = jnp.sum(oh, axis=0)[None, :]

    @pl.when(i == 0)
    def _():
      o_ref[...] = jnp.zeros_like(o_ref)
      c_ref[...] = jnp.zeros_like(c_ref)

    o_ref[...] += ps
    c_ref[...] += pc

  return pl.pallas_call(
      body,
      grid=(G,),
      in_specs=[
          pl.BlockSpec((1, 1, TBLK), lambda i: (i, 0, 0)),
          pl.BlockSpec((TBLK, D), lambda i: (i, 0)),
      ],
      out_specs=[
          pl.BlockSpec((NSEG, D), lambda i: (0, 0)),
          pl.BlockSpec((1, NSEG), lambda i: (0, 0)),
      ],
      out_shape=(
          jax.ShapeDtypeStruct((NSEG, D), jnp.float32),
          jax.ShapeDtypeStruct((1, NSEG), jnp.float32),
      ),
  )(batch_tc, x)


def _combine(sums, cnts, tsum, tcnt):
  def body(s_ref, c_ref, ts_ref, tc_ref, o_ref):
    ssum = jnp.sum(s_ref[...], axis=0) + ts_ref[...]
    cc = jnp.sum(c_ref[...][:, :, 0], axis=0) + tc_ref[0]
    o_ref[...] = ssum / jnp.maximum(cc, 1.0)[:, None]

  return pl.pallas_call(
      body,
      out_shape=jax.ShapeDtypeStruct((NSEG, D), jnp.float32),
  )(sums, cnts, tsum, tcnt)


@jax.jit
def kernel(x, batch):
  batch = batch.astype(jnp.int32)
  batch_tc = batch[:NTC].reshape(G, 1, TBLK)
  tsum, tcnt = _tc_segment_sums(x, batch_tc)
  sums, cnts = _sc_segment_sums(x, batch)
  return _combine(sums.reshape(NW, NSEG, D), cnts.reshape(NW, NSEG, 16),
                  tsum, tcnt)


# speedup vs baseline: 1.6454x; 1.0727x over previous
import functools

import jax
import jax.numpy as jnp
from jax import lax
from jax.experimental import pallas as pl
from jax.experimental.pallas import tpu as pltpu
from jax.experimental.pallas import tpu_sc as plsc

N_ROWS = 50000
D = 256
NSEG = 128

TBLK = 2000
NTC = 34000
G = NTC // TBLK

BASE = NTC
NSC = N_ROWS - NTC
CHUNK = 80
NBLK = NSC // CHUNK
NC = 2
NS = 16
NW = NC * NS
SUMW = NSEG * D
CNTW = NSEG * 16
MAXB = NBLK // NW + 1
NBUF = 4


def _sc_segment_sums(x, batch):
  mesh = plsc.VectorSubcoreMesh(core_axis_name="c", subcore_axis_name="s")

  @functools.partial(
      pl.kernel,
      mesh=mesh,
      out_type=(
          jax.ShapeDtypeStruct((NW, SUMW), jnp.float32),
          jax.ShapeDtypeStruct((NW, CNTW), jnp.float32),
      ),
      scratch_types=[
          pltpu.VMEM((MAXB * CHUNK,), jnp.int32),
          *[pltpu.VMEM((CHUNK, D), jnp.float32) for _ in range(NBUF)],
          pltpu.VMEM((SUMW,), jnp.float32),
          pltpu.VMEM((CNTW,), jnp.float32),
          *[pltpu.SemaphoreType.DMA for _ in range(NBUF)],
          pltpu.SemaphoreType.DMA,
      ],
  )
  def k(x_hbm, b_hbm, sums_hbm, cnts_hbm,
        idx_all, xb0, xb1, xb2, xb3, acc_v, cnt_v,
        sem0, sem1, sem2, sem3, semi):
    bufs = (xb0, xb1, xb2, xb3)
    sems = (sem0, sem1, sem2, sem3)
    c = lax.axis_index("c")
    s = lax.axis_index("s")
    w = c * NS + s

    one16 = jnp.ones((16,), jnp.float32)
    zero16 = jnp.zeros((16,), jnp.float32)
    lo = w * NBLK // NW
    hi = (w + 1) * NBLK // NW

    pltpu.async_copy(
        b_hbm.at[pl.ds(BASE + lo * CHUNK, MAXB * CHUNK)], idx_all, semi)

    def issue(b, xbuf, sem):
      pltpu.async_copy(x_hbm.at[pl.ds(BASE + b * CHUNK, CHUNK)], xbuf, sem)

    def drain(xbuf, sem):
      pltpu.make_async_copy(x_hbm.at[pl.ds(0, CHUNK)], xbuf, sem).wait()

    for kk in range(NBUF):
      issue(lo + kk, bufs[kk], sems[kk])

    def zsum_body(i, carry):
      for u in range(8):
        acc_v[pl.ds(i * 128 + u * 16, 16)] = zero16
      return carry

    lax.fori_loop(0, SUMW // 128, zsum_body, 0)

    def zcnt_body(i, carry):
      for u in range(8):
        cnt_v[pl.ds(i * 128 + u * 16, 16)] = zero16
      return carry

    lax.fori_loop(0, CNTW // 128, zcnt_body, 0)

    pltpu.make_async_copy(b_hbm.at[pl.ds(0, MAXB * CHUNK)], idx_all, semi
                          ).wait()

    def compute(b, xbuf):
      ioff = (b - lo) * CHUNK

      def group_body(g, carry2):
        segs = idx_all[pl.ds(ioff + g * 16, 16)]
        for jj in range(0, 16, 2):
          sa = segs[jj]
          sb = segs[jj + 1]
          offa = sa * D
          offb = sb * D
          ra = g * 16 + jj
          rb = ra + 1
          va = [xbuf[ra, pl.ds(j * 16, 16)] for j in range(D // 16)]
          vb = [xbuf[rb, pl.ds(j * 16, 16)] for j in range(D // 16)]
          for j in range(D // 16):
            plsc.addupdate(acc_v.at[pl.ds(offa + j * 16, 16)], va[j])
          plsc.addupdate(cnt_v.at[pl.ds(sa * 16, 16)], one16)
          for j in range(D // 16):
            plsc.addupdate(acc_v.at[pl.ds(offb + j * 16, 16)], vb[j])
          plsc.addupdate(cnt_v.at[pl.ds(sb * 16, 16)], one16)
        return carry2

      lax.fori_loop(0, CHUNK // 16, group_body, 0)

    def ring_body(q, carry):
      for kk in range(NBUF):
        b = lo + q * NBUF + kk

        @pl.when(b < hi)
        def _():
          drain(bufs[kk], sems[kk])
          compute(b, bufs[kk])

          @pl.when(b + NBUF < hi)
          def _():
            issue(b + NBUF, bufs[kk], sems[kk])

      return carry

    lax.fori_loop(0, (MAXB + NBUF - 1) // NBUF, ring_body, 0)

    pltpu.sync_copy(acc_v, sums_hbm.at[w])
    pltpu.sync_copy(cnt_v, cnts_hbm.at[w])

  return k(x, batch)


def _tc_segment_sums(x, batch_tc):
  def body(b_ref, x_ref, o_ref, c_ref):
    i = pl.program_id(0)
    seg = b_ref[0, 0, :]
    oh = (seg[:, None]
          == lax.broadcasted_iota(jnp.int32, (TBLK, NSEG), 1)
          ).astype(jnp.float32)
    ps = lax.dot_general(oh, x_ref[...], (((0,), (0,)), ((), ())),
                         preferred_element_type=jnp.float32)
    pc = jnp.sum(oh, axis=0)[None, :]

    @pl.when(i == 0)
    def _():
      o_ref[...] = jnp.zeros_like(o_ref)
      c_ref[...] = jnp.zeros_like(c_ref)

    o_ref[...] += ps
    c_ref[...] += pc

  return pl.pallas_call(
      body,
      grid=(G,),
      in_specs=[
          pl.BlockSpec((1, 1, TBLK), lambda i: (i, 0, 0)),
          pl.BlockSpec((TBLK, D), lambda i: (i, 0)),
      ],
      out_specs=[
          pl.BlockSpec((NSEG, D), lambda i: (0, 0)),
          pl.BlockSpec((1, NSEG), lambda i: (0, 0)),
      ],
      out_shape=(
          jax.ShapeDtypeStruct((NSEG, D), jnp.float32),
          jax.ShapeDtypeStruct((1, NSEG), jnp.float32),
      ),
  )(batch_tc, x)


def _combine(sums, cnts, tsum, tcnt):
  def body(s_ref, c_ref, ts_ref, tc_ref, o_ref):
    ssum = jnp.sum(s_ref[...], axis=0) + ts_ref[...]
    cc = jnp.sum(c_ref[...][:, :, 0], axis=0) + tc_ref[0]
    o_ref[...] = ssum / jnp.maximum(cc, 1.0)[:, None]

  return pl.pallas_call(
      body,
      out_shape=jax.ShapeDtypeStruct((NSEG, D), jnp.float32),
  )(sums, cnts, tsum, tcnt)


@jax.jit
def kernel(x, batch):
  batch = batch.astype(jnp.int32)
  batch_tc = batch[:NTC].reshape(G, 1, TBLK)
  tsum, tcnt = _tc_segment_sums(x, batch_tc)
  sums, cnts = _sc_segment_sums(x, batch)
  return _combine(sums.reshape(NW, NSEG, D), cnts.reshape(NW, NSEG, 16),
                  tsum, tcnt)
